# Initial kernel scaffold; baseline (speedup 1.0000x reference)
import jax
import jax.numpy as jnp
from jax.experimental import pallas as pl


def kernel(Q, K, V):
    raise NotImplementedError("write your pallas kernel here")


# speedup vs baseline: 4.0358x; 4.0358x over previous
import functools
import math

import numpy as np
import jax
import jax.numpy as jnp
from jax.experimental import pallas as pl

_B = 32
_H = 16
_L = 256
_DM = 4096
_K = int(2 * math.log(_L))

_t = np.arange(_L)
_ang = 2.0 * np.pi * np.outer(_t, _t) / _L
_COS = np.cos(_ang).astype(np.float32)
_SIN = np.sin(_ang).astype(np.float32)


def _corr_topk_kernel(cos_ref, sin_ref, q_ref, k_ref, v_ref, o_ref):
    A = cos_ref[...]
    S = sin_ref[...]
    q = q_ref[0]
    k = k_ref[0]
    v = v_ref[0]

    dot = functools.partial(jnp.dot, preferred_element_type=jnp.float32,
                            precision=jax.lax.Precision.HIGHEST)
    qr = dot(A, q)
    qi = dot(S, q)
    kr = dot(A, k)
    ki = dot(S, k)
    pr = qr * kr + qi * ki
    pi = qr * ki - qi * kr
    corr = (dot(A, pr) - dot(S, pi)) * (1.0 / _L)

    riota = jax.lax.broadcasted_iota(jnp.int32, (_L, _L), 0)
    diota = jax.lax.broadcasted_iota(jnp.int32, (1, _L), 1)

    c = corr
    m0 = None
    num = jnp.zeros((1, _L), dtype=jnp.float32)
    den = jnp.zeros((1, _L), dtype=jnp.float32)
    neg_inf = jnp.float32(-jnp.inf)
    for i in range(_K):
        m = jnp.max(c, axis=0, keepdims=True)
        is_m = c == m
        pos = jnp.min(jnp.where(is_m, riota, _L), axis=0, keepdims=True)
        sel = riota == pos
        c = jnp.where(sel, neg_inf, c)
        if i == 0:
            m0 = m
            e = jnp.ones((1, _L), dtype=jnp.float32)
        else:
            e = jnp.exp(m - m0)
        tgt = jnp.minimum(pos + diota, _L - 1)
        onehot = riota == tgt
        g = jnp.sum(jnp.where(onehot, v, 0.0), axis=0, keepdims=True)
        num = num + e * g
        den = den + e

    out_row = (2.0 * _L) * num / den
    o_ref[0] = jnp.broadcast_to(out_row, (_L, _L))


def kernel(Q, K, V):
    grid = (_B, _H)
    bh_spec = pl.BlockSpec((1, _L, _L), lambda b, h: (b, 0, h))
    const_spec = pl.BlockSpec((_L, _L), lambda b, h: (0, 0))
    out = pl.pallas_call(
        _corr_topk_kernel,
        grid=grid,
        in_specs=[const_spec, const_spec, bh_spec, bh_spec, bh_spec],
        out_specs=bh_spec,
        out_shape=jax.ShapeDtypeStruct((_B, _L, _DM), jnp.float32),
    )(jnp.asarray(_COS), jnp.asarray(_SIN), Q, K, V)
    return out
